# Initial kernel scaffold; baseline (speedup 1.0000x reference)
import jax
import jax.numpy as jnp
from jax.experimental import pallas as pl


def kernel(subs, rels, node_batch, node_ent, edge_src, edge_rel, edge_dst, ent_embed, rel_embed, gamma, W_h, Wq, Wm, w_alpha, gru_W_ih, gru_W_hh, gru_b_ih, gru_b_hh, W_logic, W_struct):
    raise NotImplementedError("write your pallas kernel here")


# speedup vs baseline: 1.1040x; 1.1040x over previous
import jax
import jax.numpy as jnp
from jax.experimental import pallas as pl

_BE = 2560
_BN = 8000


def _edge_kernel(bidx_ref, hsrc_ref, rele_ref, qrel_ref, wq_ref, wm_ref,
                 walpha_ref, wh_ref, out_ref):
    msg = hsrc_ref[...] + rele_ref[...]
    bidx = bidx_ref[0]
    oh = (bidx == jax.lax.broadcasted_iota(
        jnp.int32, (bidx.shape[0], qrel_ref.shape[0]), 1)).astype(jnp.float32)
    qw = jnp.dot(qrel_ref[...], wq_ref[...],
                 preferred_element_type=jnp.float32)
    pre = jnp.dot(oh, qw, preferred_element_type=jnp.float32) \
        + jnp.dot(msg, wm_ref[...], preferred_element_type=jnp.float32)
    pre = jnp.maximum(pre, 0.0)
    attn = jax.nn.sigmoid(jnp.dot(pre, walpha_ref[...],
                                  preferred_element_type=jnp.float32))
    out_ref[...] = attn * jnp.dot(msg, wh_ref[...],
                                  preferred_element_type=jnp.float32)


def _gru_kernel(agg_ref, h_ref, wih_ref, whh_ref, bih_ref, bhh_ref, out_ref):
    x = jnp.maximum(agg_ref[...], 0.0)
    h = h_ref[...]
    gi = jax.lax.dot_general(x, wih_ref[...], (((1,), (1,)), ((), ())),
                             preferred_element_type=jnp.float32) + bih_ref[...]
    gh = jax.lax.dot_general(h, whh_ref[...], (((1,), (1,)), ((), ())),
                             preferred_element_type=jnp.float32) + bhh_ref[...]
    d = h.shape[1]
    r = jax.nn.sigmoid(gi[:, :d] + gh[:, :d])
    z = jax.nn.sigmoid(gi[:, d:2 * d] + gh[:, d:2 * d])
    n = jnp.tanh(gi[:, 2 * d:] + r * gh[:, 2 * d:])
    out_ref[...] = (1.0 - z) * n + z * h


def _score_kernel(nb_ref, hop1_ref, hop2_ref, hop3_ref, tgt_ref, qrel_ref,
                  sub_ref, wstruct_ref, wl1_ref, wl2_ref, wl3_ref, out_ref):
    q_sem = jax.lax.dot_general(sub_ref[...] * qrel_ref[...], wstruct_ref[...],
                                (((1,), (1,)), ((), ())),
                                preferred_element_type=jnp.float32)
    p = jax.lax.dot_general(tgt_ref[...], q_sem, (((1,), (1,)), ((), ())),
                            preferred_element_type=jnp.float32)
    nb = nb_ref[0]
    oh = (nb == jax.lax.broadcasted_iota(
        jnp.int32, (nb.shape[0], q_sem.shape[0]), 1)).astype(jnp.float32)
    s_struct = jnp.sum(p * oh, axis=1, keepdims=True)
    s = jnp.dot(hop1_ref[...], wl1_ref[...], preferred_element_type=jnp.float32)
    s += jnp.dot(hop2_ref[...], wl2_ref[...], preferred_element_type=jnp.float32)
    s += jnp.dot(hop3_ref[...], wl3_ref[...], preferred_element_type=jnp.float32)
    out_ref[...] = s + s_struct


def kernel(subs, rels, node_batch, node_ent, edge_src, edge_rel, edge_dst,
           ent_embed, rel_embed, gamma, W_h, Wq, Wm, w_alpha,
           gru_W_ih, gru_W_hh, gru_b_ih, gru_b_hh, W_logic, W_struct):
    n_ent, d = ent_embed.shape
    bsz = subs.shape[0]
    n_nodes = node_batch.shape[0]
    n_layer = gamma.shape[0]
    n_edge = edge_src.shape[0]
    a = Wq.shape[2]
    ge = n_edge // _BE
    gn = n_nodes // _BN

    q_rel = rel_embed[rels]
    sub_embed = ent_embed[subs]
    rel_e = rel_embed[edge_rel]
    bidx_e = node_batch[edge_dst].astype(jnp.int32).reshape(ge, _BE, 1)
    gamma_norm = jax.nn.softmax(gamma)

    full_spec = lambda shape: pl.BlockSpec(shape, lambda i: (0,) * len(shape))
    edge_call = pl.pallas_call(
        _edge_kernel,
        grid=(ge,),
        in_specs=[
            pl.BlockSpec((1, _BE, 1), lambda i: (i, 0, 0)),
            pl.BlockSpec((_BE, d), lambda i: (i, 0)),
            pl.BlockSpec((_BE, d), lambda i: (i, 0)),
            full_spec((bsz, d)),
            full_spec((d, a)),
            full_spec((d, a)),
            full_spec((a, 1)),
            full_spec((d, d)),
        ],
        out_specs=pl.BlockSpec((_BE, d), lambda i: (i, 0)),
        out_shape=jax.ShapeDtypeStruct((n_edge, d), jnp.float32),
    )

    gru_call = pl.pallas_call(
        _gru_kernel,
        grid=(gn,),
        in_specs=[
            pl.BlockSpec((_BN, d), lambda i: (i, 0)),
            pl.BlockSpec((_BN, d), lambda i: (i, 0)),
            full_spec((3 * d, d)),
            full_spec((3 * d, d)),
            full_spec((1, 3 * d)),
            full_spec((1, 3 * d)),
        ],
        out_specs=pl.BlockSpec((_BN, d), lambda i: (i, 0)),
        out_shape=jax.ShapeDtypeStruct((n_nodes, d), jnp.float32),
    )

    seed_table = jnp.concatenate(
        [sub_embed, jnp.zeros((1, d), jnp.float32)], axis=0)
    small_idx = jnp.where(edge_src < bsz, edge_src, bsz)

    h_state = jnp.zeros((n_nodes, d), jnp.float32)
    hidden = None
    hops = []
    for i in range(n_layer):
        if i == 0:
            hsrc = seed_table[small_idx]
        else:
            hsrc = hidden[edge_src]
        wmsg = edge_call(bidx_e, hsrc, rel_e, q_rel, Wq[i], Wm[i],
                         w_alpha[i].reshape(a, 1), W_h[i])
        agg = jnp.zeros((n_nodes, d), jnp.float32).at[edge_dst].add(wmsg)
        h_state = gru_call(agg, h_state, gru_W_ih, gru_W_hh,
                           gru_b_ih.reshape(1, 3 * d),
                           gru_b_hh.reshape(1, 3 * d))
        hidden = h_state
        hops.append(hidden)

    tgt = ent_embed[node_ent]
    nb3 = node_batch.astype(jnp.int32).reshape(gn, _BN, 1)
    wl = [(gamma_norm[i] * W_logic).reshape(d, 1) for i in range(n_layer)]
    score_call = pl.pallas_call(
        _score_kernel,
        grid=(gn,),
        in_specs=[
            pl.BlockSpec((1, _BN, 1), lambda i: (i, 0, 0)),
            pl.BlockSpec((_BN, d), lambda i: (i, 0)),
            pl.BlockSpec((_BN, d), lambda i: (i, 0)),
            pl.BlockSpec((_BN, d), lambda i: (i, 0)),
            pl.BlockSpec((_BN, d), lambda i: (i, 0)),
            full_spec((bsz, d)),
            full_spec((bsz, d)),
            full_spec((d, d)),
            full_spec((d, 1)),
            full_spec((d, 1)),
            full_spec((d, 1)),
        ],
        out_specs=pl.BlockSpec((_BN, 1), lambda i: (i, 0)),
        out_shape=jax.ShapeDtypeStruct((n_nodes, 1), jnp.float32),
    )
    v = score_call(nb3, hops[0], hops[1], hops[2], tgt, q_rel, sub_embed,
                   W_struct, wl[0], wl[1], wl[2])[:, 0]

    full_scores = jnp.zeros((bsz, n_ent), jnp.float32)
    full_scores = full_scores.at[node_batch, node_ent].add(v)
    return full_scores
